# Initial kernel scaffold; baseline (speedup 1.0000x reference)
import jax
import jax.numpy as jnp
from jax.experimental import pallas as pl


def kernel(preds, ground_truth):
    raise NotImplementedError("write your pallas kernel here")


# speedup vs baseline: 80.1002x; 80.1002x over previous
import numpy as np
import jax
import jax.numpy as jnp
from jax.experimental import pallas as pl

_NV = 512
_NC = 64
_AP = 8
_B = 16384
_K = 128
_T = 1024
_G = _B // _T


def _build_tables():
    rng = np.random.default_rng(42)
    cons = []
    for c in range(_NC):
        pool = np.delete(np.arange(_NV), c)
        others = rng.choice(pool, size=_AP - 1, replace=False)
        body = [(int(c), float(rng.uniform(0.5, 1.5)), bool(rng.integers(0, 2)))]
        for v in others:
            body.append((int(v), float(rng.uniform(0.5, 1.5)),
                         bool(rng.integers(0, 2))))
        cons.append(body)
    masks = [b[1][0] for b in cons]
    atoms = [[(v, co * (-1.0 if s else 1.0)) for (v, co, s) in body[2:]]
             for body in cons]

    E = np.zeros((_NV, _K), np.float32)
    W = np.zeros((_NV, _K), np.float32)
    AT = np.zeros((_K, _K), np.float32)
    M = np.zeros((_K, _K), np.float32)
    G0 = np.ones((1, _K), np.float32)
    GP = np.zeros((_K, _K), np.float32)
    single_idx = {}
    k = 0
    for c in range(_NC):
        m = masks[c]
        if m < c:
            pk = single_idx[m]
            E[masks[m], k] = 1.0
            for v, w in atoms[m]:
                W[v, k] += w
            for v, w in atoms[c]:
                W[v, k] += w
            AT[c, k] = 1.0
            M[k, c] = 1.0
            G0[0, k] = 0.0
            GP[pk, k] = 1.0
            k += 1
            E[m, k] = 1.0
            for v, w in atoms[c]:
                W[v, k] += w
            AT[c, k] = 1.0
            M[k, c] = 1.0
            GP[pk, k] = -1.0
            k += 1
        else:
            E[m, k] = 1.0
            for v, w in atoms[c]:
                W[v, k] += w
            AT[c, k] = 1.0
            M[k, c] = 1.0
            single_idx[c] = k
            k += 1
    return E, W, AT, M, G0, GP


_E, _W, _AT, _M, _G0, _GP = _build_tables()


def _pass1(p_ref, g_ref, e_ref, w_ref, out_ref, cand_ref, ssc_ref, ssa_ref):
    i = pl.program_id(0)
    p = p_ref[:, :]
    cand = (jnp.dot(p, e_ref[:, :], preferred_element_type=jnp.float32)
            + jnp.dot(g_ref[:, :], w_ref[:, :],
                      preferred_element_type=jnp.float32))
    out_ref[:, :] = p
    cand_ref[:, :] = cand
    ssc = jnp.sum(cand * cand, axis=0, keepdims=True)
    pa = p[:, :_K]
    ssa = jnp.sum(pa * pa, axis=0, keepdims=True)

    @pl.when(i == 0)
    def _():
        ssc_ref[:, :] = ssc
        ssa_ref[:, :] = ssa

    @pl.when(i != 0)
    def _():
        ssc_ref[:, :] = ssc_ref[:, :] + ssc
        ssa_ref[:, :] = ssa_ref[:, :] + ssa


def _pass2(p_ref, cand_ref, ssc_ref, ssa_ref, at_ref, m_ref, g0_ref, gp_ref,
           prev_ref, out_ref):
    del prev_ref
    t = jnp.dot(ssa_ref[:, :], at_ref[:, :], preferred_element_type=jnp.float32)
    raw = (ssc_ref[:, :] > t).astype(jnp.float32)
    gate = g0_ref[:, :] + jnp.dot(raw, gp_ref[:, :],
                                  preferred_element_type=jnp.float32)
    sel = gate * raw
    fired = jnp.dot(sel, m_ref[:, :], preferred_element_type=jnp.float32)
    contrib = jnp.dot(cand_ref[:, :] * sel, m_ref[:, :],
                      preferred_element_type=jnp.float32)
    out_ref[:, :] = p_ref[:, :] * (1.0 - fired) + contrib


def kernel(preds, ground_truth):
    e = jnp.asarray(_E)
    w = jnp.asarray(_W)
    at = jnp.asarray(_AT)
    m = jnp.asarray(_M)
    g0 = jnp.asarray(_G0)
    gp = jnp.asarray(_GP)

    out1, cand, ssc, ssa = pl.pallas_call(
        _pass1,
        grid=(_G,),
        in_specs=[
            pl.BlockSpec((_T, _NV), lambda i: (i, 0)),
            pl.BlockSpec((_T, _NV), lambda i: (i, 0)),
            pl.BlockSpec((_NV, _K), lambda i: (0, 0)),
            pl.BlockSpec((_NV, _K), lambda i: (0, 0)),
        ],
        out_specs=[
            pl.BlockSpec((_T, _NV), lambda i: (i, 0)),
            pl.BlockSpec((_T, _K), lambda i: (i, 0)),
            pl.BlockSpec((1, _K), lambda i: (0, 0)),
            pl.BlockSpec((1, _K), lambda i: (0, 0)),
        ],
        out_shape=[
            jax.ShapeDtypeStruct((_B, _NV), jnp.float32),
            jax.ShapeDtypeStruct((_B, _K), jnp.float32),
            jax.ShapeDtypeStruct((1, _K), jnp.float32),
            jax.ShapeDtypeStruct((1, _K), jnp.float32),
        ],
    )(preds, ground_truth, e, w)

    out = pl.pallas_call(
        _pass2,
        grid=(_G,),
        in_specs=[
            pl.BlockSpec((_T, _K), lambda i: (i, 0)),
            pl.BlockSpec((_T, _K), lambda i: (i, 0)),
            pl.BlockSpec((1, _K), lambda i: (0, 0)),
            pl.BlockSpec((1, _K), lambda i: (0, 0)),
            pl.BlockSpec((_K, _K), lambda i: (0, 0)),
            pl.BlockSpec((_K, _K), lambda i: (0, 0)),
            pl.BlockSpec((1, _K), lambda i: (0, 0)),
            pl.BlockSpec((_K, _K), lambda i: (0, 0)),
            pl.BlockSpec((8, _K), lambda i: (0, 0)),
        ],
        out_specs=pl.BlockSpec((_T, _K), lambda i: (i, 0)),
        out_shape=jax.ShapeDtypeStruct((_B, _NV), jnp.float32),
        input_output_aliases={8: 0},
    )(preds, cand, ssc, ssa, at, m, g0, gp, out1)
    return out


# speedup vs baseline: 92.6269x; 1.1564x over previous
import numpy as np
import jax
import jax.numpy as jnp
from jax.experimental import pallas as pl

_NV = 512
_NC = 64
_AP = 8
_B = 16384
_K = 128
_T = 2048
_G = _B // _T


def _build_tables():
    rng = np.random.default_rng(42)
    cons = []
    for c in range(_NC):
        pool = np.delete(np.arange(_NV), c)
        others = rng.choice(pool, size=_AP - 1, replace=False)
        body = [(int(c), float(rng.uniform(0.5, 1.5)), bool(rng.integers(0, 2)))]
        for v in others:
            body.append((int(v), float(rng.uniform(0.5, 1.5)),
                         bool(rng.integers(0, 2))))
        cons.append(body)
    masks = [b[1][0] for b in cons]
    atoms = [[(v, co * (-1.0 if s else 1.0)) for (v, co, s) in body[2:]]
             for body in cons]

    E = np.zeros((_NV, _K), np.float32)
    W = np.zeros((_NV, _K), np.float32)
    AT = np.zeros((_K, _K), np.float32)
    M = np.zeros((_K, _K), np.float32)
    G0 = np.ones((1, _K), np.float32)
    GP = np.zeros((_K, _K), np.float32)
    single_idx = {}
    k = 0
    for c in range(_NC):
        m = masks[c]
        if m < c:
            pk = single_idx[m]
            E[masks[m], k] = 1.0
            for v, w in atoms[m]:
                W[v, k] += w
            for v, w in atoms[c]:
                W[v, k] += w
            AT[c, k] = 1.0
            M[k, c] = 1.0
            G0[0, k] = 0.0
            GP[pk, k] = 1.0
            k += 1
            E[m, k] = 1.0
            for v, w in atoms[c]:
                W[v, k] += w
            AT[c, k] = 1.0
            M[k, c] = 1.0
            GP[pk, k] = -1.0
            k += 1
        else:
            E[m, k] = 1.0
            for v, w in atoms[c]:
                W[v, k] += w
            AT[c, k] = 1.0
            M[k, c] = 1.0
            single_idx[c] = k
            k += 1
    return E, W, AT, M, G0, GP


_E, _W, _AT, _M, _G0, _GP = _build_tables()


def _pass1(p_ref, g_ref, e_ref, w_ref, out_ref, cand_ref, ssc_ref, ssa_ref):
    i = pl.program_id(0)
    p = p_ref[:, :]
    cand = (jnp.dot(p, e_ref[:, :], preferred_element_type=jnp.float32)
            + jnp.dot(g_ref[:, :], w_ref[:, :],
                      preferred_element_type=jnp.float32))
    out_ref[:, :] = p
    cand_ref[:, :] = cand
    ssc = jnp.sum(cand * cand, axis=0, keepdims=True)
    pa = p[:, :_K]
    ssa = jnp.sum(pa * pa, axis=0, keepdims=True)

    @pl.when(i == 0)
    def _():
        ssc_ref[:, :] = ssc
        ssa_ref[:, :] = ssa

    @pl.when(i != 0)
    def _():
        ssc_ref[:, :] = ssc_ref[:, :] + ssc
        ssa_ref[:, :] = ssa_ref[:, :] + ssa


def _pass2(p_ref, cand_ref, ssc_ref, ssa_ref, at_ref, m_ref, g0_ref, gp_ref,
           prev_ref, out_ref):
    del prev_ref
    t = jnp.dot(ssa_ref[:, :], at_ref[:, :], preferred_element_type=jnp.float32)
    raw = (ssc_ref[:, :] > t).astype(jnp.float32)
    gate = g0_ref[:, :] + jnp.dot(raw, gp_ref[:, :],
                                  preferred_element_type=jnp.float32)
    sel = gate * raw
    fired = jnp.dot(sel, m_ref[:, :], preferred_element_type=jnp.float32)
    contrib = jnp.dot(cand_ref[:, :] * sel, m_ref[:, :],
                      preferred_element_type=jnp.float32)
    out_ref[:, :] = p_ref[:, :] * (1.0 - fired) + contrib


def kernel(preds, ground_truth):
    e = jnp.asarray(_E)
    w = jnp.asarray(_W)
    at = jnp.asarray(_AT)
    m = jnp.asarray(_M)
    g0 = jnp.asarray(_G0)
    gp = jnp.asarray(_GP)

    out1, cand, ssc, ssa = pl.pallas_call(
        _pass1,
        grid=(_G,),
        in_specs=[
            pl.BlockSpec((_T, _NV), lambda i: (i, 0)),
            pl.BlockSpec((_T, _NV), lambda i: (i, 0)),
            pl.BlockSpec((_NV, _K), lambda i: (0, 0)),
            pl.BlockSpec((_NV, _K), lambda i: (0, 0)),
        ],
        out_specs=[
            pl.BlockSpec((_T, _NV), lambda i: (i, 0)),
            pl.BlockSpec((_T, _K), lambda i: (i, 0)),
            pl.BlockSpec((1, _K), lambda i: (0, 0)),
            pl.BlockSpec((1, _K), lambda i: (0, 0)),
        ],
        out_shape=[
            jax.ShapeDtypeStruct((_B, _NV), jnp.float32),
            jax.ShapeDtypeStruct((_B, _K), jnp.float32),
            jax.ShapeDtypeStruct((1, _K), jnp.float32),
            jax.ShapeDtypeStruct((1, _K), jnp.float32),
        ],
    )(preds, ground_truth, e, w)

    out = pl.pallas_call(
        _pass2,
        grid=(_G,),
        in_specs=[
            pl.BlockSpec((_T, _K), lambda i: (i, 0)),
            pl.BlockSpec((_T, _K), lambda i: (i, 0)),
            pl.BlockSpec((1, _K), lambda i: (0, 0)),
            pl.BlockSpec((1, _K), lambda i: (0, 0)),
            pl.BlockSpec((_K, _K), lambda i: (0, 0)),
            pl.BlockSpec((_K, _K), lambda i: (0, 0)),
            pl.BlockSpec((1, _K), lambda i: (0, 0)),
            pl.BlockSpec((_K, _K), lambda i: (0, 0)),
            pl.BlockSpec((8, _K), lambda i: (0, 0)),
        ],
        out_specs=pl.BlockSpec((_T, _K), lambda i: (i, 0)),
        out_shape=jax.ShapeDtypeStruct((_B, _NV), jnp.float32),
        input_output_aliases={8: 0},
    )(preds, cand, ssc, ssa, at, m, g0, gp, out1)
    return out


# speedup vs baseline: 97.7597x; 1.0554x over previous
import numpy as np
import jax
import jax.numpy as jnp
from jax.experimental import pallas as pl

_NV = 512
_NC = 64
_AP = 8
_B = 16384
_K = 128
_T = 4096
_G = _B // _T


def _build_tables():
    rng = np.random.default_rng(42)
    cons = []
    for c in range(_NC):
        pool = np.delete(np.arange(_NV), c)
        others = rng.choice(pool, size=_AP - 1, replace=False)
        body = [(int(c), float(rng.uniform(0.5, 1.5)), bool(rng.integers(0, 2)))]
        for v in others:
            body.append((int(v), float(rng.uniform(0.5, 1.5)),
                         bool(rng.integers(0, 2))))
        cons.append(body)
    masks = [b[1][0] for b in cons]
    atoms = [[(v, co * (-1.0 if s else 1.0)) for (v, co, s) in body[2:]]
             for body in cons]

    E = np.zeros((_NV, _K), np.float32)
    W = np.zeros((_NV, _K), np.float32)
    AT = np.zeros((_K, _K), np.float32)
    M = np.zeros((_K, _K), np.float32)
    G0 = np.ones((1, _K), np.float32)
    GP = np.zeros((_K, _K), np.float32)
    single_idx = {}
    k = 0
    for c in range(_NC):
        m = masks[c]
        if m < c:
            pk = single_idx[m]
            E[masks[m], k] = 1.0
            for v, w in atoms[m]:
                W[v, k] += w
            for v, w in atoms[c]:
                W[v, k] += w
            AT[c, k] = 1.0
            M[k, c] = 1.0
            G0[0, k] = 0.0
            GP[pk, k] = 1.0
            k += 1
            E[m, k] = 1.0
            for v, w in atoms[c]:
                W[v, k] += w
            AT[c, k] = 1.0
            M[k, c] = 1.0
            GP[pk, k] = -1.0
            k += 1
        else:
            E[m, k] = 1.0
            for v, w in atoms[c]:
                W[v, k] += w
            AT[c, k] = 1.0
            M[k, c] = 1.0
            single_idx[c] = k
            k += 1
    return E, W, AT, M, G0, GP


_E, _W, _AT, _M, _G0, _GP = _build_tables()


def _pass1(p_ref, g_ref, e_ref, w_ref, out_ref, cand_ref, ssc_ref, ssa_ref):
    i = pl.program_id(0)
    p = p_ref[:, :]
    cand = (jnp.dot(p, e_ref[:, :], preferred_element_type=jnp.float32)
            + jnp.dot(g_ref[:, :], w_ref[:, :],
                      preferred_element_type=jnp.float32))
    out_ref[:, :] = p
    cand_ref[:, :] = cand
    ssc = jnp.sum(cand * cand, axis=0, keepdims=True)
    pa = p[:, :_K]
    ssa = jnp.sum(pa * pa, axis=0, keepdims=True)

    @pl.when(i == 0)
    def _():
        ssc_ref[:, :] = ssc
        ssa_ref[:, :] = ssa

    @pl.when(i != 0)
    def _():
        ssc_ref[:, :] = ssc_ref[:, :] + ssc
        ssa_ref[:, :] = ssa_ref[:, :] + ssa


def _pass2(p_ref, cand_ref, ssc_ref, ssa_ref, at_ref, m_ref, g0_ref, gp_ref,
           prev_ref, out_ref):
    del prev_ref
    t = jnp.dot(ssa_ref[:, :], at_ref[:, :], preferred_element_type=jnp.float32)
    raw = (ssc_ref[:, :] > t).astype(jnp.float32)
    gate = g0_ref[:, :] + jnp.dot(raw, gp_ref[:, :],
                                  preferred_element_type=jnp.float32)
    sel = gate * raw
    fired = jnp.dot(sel, m_ref[:, :], preferred_element_type=jnp.float32)
    contrib = jnp.dot(cand_ref[:, :] * sel, m_ref[:, :],
                      preferred_element_type=jnp.float32)
    out_ref[:, :] = p_ref[:, :] * (1.0 - fired) + contrib


def kernel(preds, ground_truth):
    e = jnp.asarray(_E)
    w = jnp.asarray(_W)
    at = jnp.asarray(_AT)
    m = jnp.asarray(_M)
    g0 = jnp.asarray(_G0)
    gp = jnp.asarray(_GP)

    out1, cand, ssc, ssa = pl.pallas_call(
        _pass1,
        grid=(_G,),
        in_specs=[
            pl.BlockSpec((_T, _NV), lambda i: (i, 0)),
            pl.BlockSpec((_T, _NV), lambda i: (i, 0)),
            pl.BlockSpec((_NV, _K), lambda i: (0, 0)),
            pl.BlockSpec((_NV, _K), lambda i: (0, 0)),
        ],
        out_specs=[
            pl.BlockSpec((_T, _NV), lambda i: (i, 0)),
            pl.BlockSpec((_T, _K), lambda i: (i, 0)),
            pl.BlockSpec((1, _K), lambda i: (0, 0)),
            pl.BlockSpec((1, _K), lambda i: (0, 0)),
        ],
        out_shape=[
            jax.ShapeDtypeStruct((_B, _NV), jnp.float32),
            jax.ShapeDtypeStruct((_B, _K), jnp.float32),
            jax.ShapeDtypeStruct((1, _K), jnp.float32),
            jax.ShapeDtypeStruct((1, _K), jnp.float32),
        ],
    )(preds, ground_truth, e, w)

    out = pl.pallas_call(
        _pass2,
        grid=(_G,),
        in_specs=[
            pl.BlockSpec((_T, _K), lambda i: (i, 0)),
            pl.BlockSpec((_T, _K), lambda i: (i, 0)),
            pl.BlockSpec((1, _K), lambda i: (0, 0)),
            pl.BlockSpec((1, _K), lambda i: (0, 0)),
            pl.BlockSpec((_K, _K), lambda i: (0, 0)),
            pl.BlockSpec((_K, _K), lambda i: (0, 0)),
            pl.BlockSpec((1, _K), lambda i: (0, 0)),
            pl.BlockSpec((_K, _K), lambda i: (0, 0)),
            pl.BlockSpec((8, _K), lambda i: (0, 0)),
        ],
        out_specs=pl.BlockSpec((_T, _K), lambda i: (i, 0)),
        out_shape=jax.ShapeDtypeStruct((_B, _NV), jnp.float32),
        input_output_aliases={8: 0},
    )(preds, cand, ssc, ssa, at, m, g0, gp, out1)
    return out


# speedup vs baseline: 105.0578x; 1.0747x over previous
import numpy as np
import jax
import jax.numpy as jnp
from jax.experimental import pallas as pl
from jax.experimental.pallas import tpu as pltpu

_NV = 512
_NC = 64
_AP = 8
_B = 16384
_K = 128
_T = 1024
_G = _B // _T


def _build_tables():
    rng = np.random.default_rng(42)
    cons = []
    for c in range(_NC):
        pool = np.delete(np.arange(_NV), c)
        others = rng.choice(pool, size=_AP - 1, replace=False)
        body = [(int(c), float(rng.uniform(0.5, 1.5)), bool(rng.integers(0, 2)))]
        for v in others:
            body.append((int(v), float(rng.uniform(0.5, 1.5)),
                         bool(rng.integers(0, 2))))
        cons.append(body)
    masks = [b[1][0] for b in cons]
    atoms = [[(v, co * (-1.0 if s else 1.0)) for (v, co, s) in body[2:]]
             for body in cons]

    E = np.zeros((_NV, _K), np.float32)
    W = np.zeros((_NV, _K), np.float32)
    AT = np.zeros((_K, _K), np.float32)
    M = np.zeros((_K, _K), np.float32)
    G0 = np.ones((1, _K), np.float32)
    GP = np.zeros((_K, _K), np.float32)
    single_idx = {}
    k = 0
    for c in range(_NC):
        m = masks[c]
        if m < c:
            pk = single_idx[m]
            E[masks[m], k] = 1.0
            for v, w in atoms[m]:
                W[v, k] += w
            for v, w in atoms[c]:
                W[v, k] += w
            AT[c, k] = 1.0
            M[k, c] = 1.0
            G0[0, k] = 0.0
            GP[pk, k] = 1.0
            k += 1
            E[m, k] = 1.0
            for v, w in atoms[c]:
                W[v, k] += w
            AT[c, k] = 1.0
            M[k, c] = 1.0
            GP[pk, k] = -1.0
            k += 1
        else:
            E[m, k] = 1.0
            for v, w in atoms[c]:
                W[v, k] += w
            AT[c, k] = 1.0
            M[k, c] = 1.0
            single_idx[c] = k
            k += 1
    return E, W, AT, M, G0, GP


_E, _W, _AT, _M, _G0, _GP = _build_tables()


def _fused(p_ref, g_ref, e_ref, w_ref, at_ref, m_ref, g0_ref, gp_ref,
           out_ref, stash_p, stash_c, ssc_s, ssa_s):
    ph = pl.program_id(0)
    i = pl.program_id(1)

    @pl.when(ph == 0)
    def _():
        p = p_ref[:, :]
        cand = (jnp.dot(p, e_ref[:, :], preferred_element_type=jnp.float32)
                + jnp.dot(g_ref[:, :], w_ref[:, :],
                          preferred_element_type=jnp.float32))
        stash_p[pl.ds(i * _T, _T), :] = p
        stash_c[pl.ds(i * _T, _T), :] = cand
        ssc = jnp.sum(cand * cand, axis=0, keepdims=True)
        pa = p[:, :_K]
        ssa = jnp.sum(pa * pa, axis=0, keepdims=True)

        @pl.when(i == 0)
        def _():
            ssc_s[:, :] = ssc
            ssa_s[:, :] = ssa

        @pl.when(i != 0)
        def _():
            ssc_s[:, :] = ssc_s[:, :] + ssc
            ssa_s[:, :] = ssa_s[:, :] + ssa

    @pl.when(ph == 1)
    def _():
        t = jnp.dot(ssa_s[:, :], at_ref[:, :],
                    preferred_element_type=jnp.float32)
        raw = (ssc_s[:, :] > t).astype(jnp.float32)
        gate = g0_ref[:, :] + jnp.dot(raw, gp_ref[:, :],
                                      preferred_element_type=jnp.float32)
        sel = gate * raw
        fired = jnp.dot(sel, m_ref[:, :], preferred_element_type=jnp.float32)
        p = stash_p[pl.ds(i * _T, _T), :]
        cand = stash_c[pl.ds(i * _T, _T), :]
        contrib = jnp.dot(cand * sel, m_ref[:, :],
                          preferred_element_type=jnp.float32)
        out_ref[:, :_K] = p[:, :_K] * (1.0 - fired) + contrib
        out_ref[:, _K:] = p[:, _K:]


def kernel(preds, ground_truth):
    e = jnp.asarray(_E)
    w = jnp.asarray(_W)
    at = jnp.asarray(_AT)
    m = jnp.asarray(_M)
    g0 = jnp.asarray(_G0)
    gp = jnp.asarray(_GP)

    return pl.pallas_call(
        _fused,
        grid=(2, _G),
        in_specs=[
            pl.BlockSpec((_T, _NV), lambda ph, i: ((1 - ph) * i, 0)),
            pl.BlockSpec((_T, _NV), lambda ph, i: ((1 - ph) * i, 0)),
            pl.BlockSpec((_NV, _K), lambda ph, i: (0, 0)),
            pl.BlockSpec((_NV, _K), lambda ph, i: (0, 0)),
            pl.BlockSpec((_K, _K), lambda ph, i: (0, 0)),
            pl.BlockSpec((_K, _K), lambda ph, i: (0, 0)),
            pl.BlockSpec((1, _K), lambda ph, i: (0, 0)),
            pl.BlockSpec((_K, _K), lambda ph, i: (0, 0)),
        ],
        out_specs=pl.BlockSpec((_T, _NV), lambda ph, i: (ph * i, 0)),
        out_shape=jax.ShapeDtypeStruct((_B, _NV), jnp.float32),
        scratch_shapes=[
            pltpu.VMEM((_B, _NV), jnp.float32),
            pltpu.VMEM((_B, _K), jnp.float32),
            pltpu.VMEM((1, _K), jnp.float32),
            pltpu.VMEM((1, _K), jnp.float32),
        ],
    )(preds, ground_truth, e, w, at, m, g0, gp)
